# baseline (speedup 1.0000x reference)
import functools

import jax
import jax.numpy as jnp
from jax import lax
from jax.experimental import pallas as pl
from jax.experimental.pallas import tpu as pltpu
from jax.experimental.pallas import tpu_sc as plsc

_D = 1024
_E = 8
_LANES = 16
_KC = _D // _LANES
_TSUB = 4
_NW = 32


def _sc_router_body(x_hbm, w_hbm, b2_hbm, out_hbm, w_v, b_v, x_v, out_v):
    nc = 2
    wid = lax.axis_index("s") * nc + lax.axis_index("c")
    tok_per_w = x_hbm.shape[0] // _NW
    nblk = tok_per_w // _LANES
    base = wid * tok_per_w

    pltpu.sync_copy(w_hbm, w_v)
    pltpu.sync_copy(b2_hbm, b_v)

    tok_iota = lax.iota(jnp.int32, _LANES)
    b2 = b_v[...]

    def blk_body(blk, _):
        tok0 = base + blk * _LANES
        pltpu.sync_copy(x_hbm.at[pl.ds(tok0, _LANES)], x_v)

        svals = [[None] * _E for _ in range(_LANES)]
        for sub in range(_LANES // _TSUB):
            def kc_body(kc, accs):
                accs = list(accs)
                for t in range(_TSUB):
                    xv = x_v[sub * _TSUB + t, pl.ds(kc * _LANES, _LANES)]
                    for e in range(_E):
                        wv = w_v[e, pl.ds(kc * _LANES, _LANES)]
                        accs[t * _E + e] = accs[t * _E + e] + xv * wv
                return tuple(accs)

            init = tuple(jnp.zeros((_LANES,), jnp.float32)
                         for _ in range(_TSUB * _E))
            accs = lax.fori_loop(0, _KC, kc_body, init)
            for t in range(_TSUB):
                for e in range(_E):
                    svals[sub * _TSUB + t][e] = jnp.sum(accs[t * _E + e])

        for p in range(_LANES // 2):
            v = jnp.zeros((_LANES,), jnp.float32)
            for e in range(_E):
                v = jnp.where(tok_iota == e, svals[2 * p][e], v)
                v = jnp.where(tok_iota == e + _E, svals[2 * p + 1][e], v)
            ev = jnp.exp(v + b2)
            cs = plsc.cumsum(ev)
            s0 = cs[_E - 1]
            s1 = cs[_LANES - 1] - s0
            out_v[p, :] = ev / jnp.where(tok_iota < _E, s0, s1)

        row0 = pl.multiple_of(tok0 // 2, _LANES // 2)
        pltpu.sync_copy(out_v, out_hbm.at[pl.ds(row0, _LANES // 2)])
        return ()

    lax.fori_loop(0, nblk, blk_body, ())


def sc_router(x, W, b2):
    n = x.shape[0]
    mesh = plsc.VectorSubcoreMesh(core_axis_name="c", subcore_axis_name="s")
    f = functools.partial(
        pl.kernel,
        mesh=mesh,
        out_type=jax.ShapeDtypeStruct((n // 2, _LANES), jnp.float32),
        scratch_types=[
            pltpu.VMEM((_E, _D), jnp.float32),
            pltpu.VMEM((_LANES,), jnp.float32),
            pltpu.VMEM((_LANES, _D), jnp.float32),
            pltpu.VMEM((_LANES // 2, _LANES), jnp.float32),
        ],
        compiler_params=pltpu.CompilerParams(needs_layout_passes=False),
    )(_sc_router_body)
    return f(x, W, b2)


def kernel(x, W, b):
    n = x.shape[0]
    b2 = jnp.concatenate([b, b])
    packed = sc_router(x, W, b2)
    return packed.reshape(n, _E)


# speedup vs baseline: 5.7100x; 5.7100x over previous
import jax
import jax.numpy as jnp
from jax.experimental import pallas as pl
from jax.experimental.pallas import tpu as pltpu

_CHUNK = 1024
_NBUF = 8


def _router_body(x_hbm, wt_ref, b_ref, out_ref, bufs, sems):
    n_chunks = x_hbm.shape[0]

    def copy_in(g, slot):
        return pltpu.make_async_copy(x_hbm.at[g], bufs.at[slot], sems.at[slot])

    for slot in range(_NBUF):
        copy_in(slot, slot).start()

    wt = wt_ref[...]
    bias = b_ref[...]

    @pl.loop(0, n_chunks, step=_NBUF)
    def outer(g0):
        for slot in range(_NBUF):
            g = g0 + slot
            copy_in(g, slot).wait()
            logits = jax.lax.dot_general(
                bufs[slot], wt, (((1,), (0,)), ((), ())),
                precision=jax.lax.Precision.DEFAULT,
                preferred_element_type=jnp.float32)
            logits = logits + bias
            m = jnp.max(logits, axis=-1, keepdims=True)
            e = jnp.exp(logits - m)
            probs = e / jnp.sum(e, axis=-1, keepdims=True)
            out_ref[pl.ds(pl.multiple_of(g * _CHUNK, _CHUNK), _CHUNK), :] = probs

            @pl.when(g + _NBUF < n_chunks)
            def _():
                copy_in(g + _NBUF, slot).start()


def kernel(x, W, b):
    n_tokens, d = x.shape
    n_experts = W.shape[0]
    x3 = x.reshape(n_tokens // _CHUNK, _CHUNK, d)
    return pl.pallas_call(
        _router_body,
        in_specs=[
            pl.BlockSpec(memory_space=pl.ANY),
            pl.BlockSpec(memory_space=pltpu.VMEM),
            pl.BlockSpec(memory_space=pltpu.VMEM),
        ],
        out_specs=pl.BlockSpec(memory_space=pltpu.VMEM),
        out_shape=jax.ShapeDtypeStruct((n_tokens, n_experts), jnp.float32),
        scratch_shapes=[
            pltpu.VMEM((_NBUF, _CHUNK, d), jnp.float32),
            pltpu.SemaphoreType.DMA((_NBUF,)),
        ],
    )(x3, W.T, b.reshape(1, n_experts))
